# baseline (device time: 15740 ns/iter reference)
import jax
import jax.numpy as jnp
from jax import lax
from jax.experimental import pallas as pl
from jax.experimental.pallas import tpu as pltpu

R = 64
K = 64


def kernel(partial, gamma):
    _, m, d = partial.shape
    half = m // 2
    chunk = half // 2
    n_mine = chunk // R
    n_late = K // R
    n_push = (chunk - K) // R
    n_y = n_mine + n_late

    def body(p3_ref, g_ref, out_ref, local_ref, comm_ref, tile0_ref,
             local_sem, tile0_sem,
             y_send_sems, y_recv_sems, x_send_sems, x_recv_sems):
        p_ref = p3_ref.at[0]
        my_x = lax.axis_index("x")
        my_y = lax.axis_index("y")
        my_z = lax.axis_index("z")
        y_partner = (my_x, 1 - my_y, my_z)
        x_partner = (1 - my_x, my_y, my_z)

        mine_half = my_y * half
        peer_half = (1 - my_y) * half
        my_chunk = my_x * chunk
        other_chunk = (1 - my_x) * chunk
        late = other_chunk + chunk - K

        def tile_row(t):
            if t < n_mine:
                return my_chunk + t * R
            return late + (t - n_mine) * R

        local_copies = [
            pltpu.make_async_copy(
                p_ref.at[pl.ds(mine_half + my_chunk, chunk), :],
                local_ref.at[pl.ds(0, chunk), :],
                local_sem,
            ),
            pltpu.make_async_copy(
                p_ref.at[pl.ds(mine_half + late, K), :],
                local_ref.at[pl.ds(chunk, K), :],
                local_sem,
            ),
        ]
        for c in local_copies:
            c.start()

        tile0_copy = pltpu.make_async_copy(
            p_ref.at[pl.ds(peer_half + tile_row(0), R), :],
            tile0_ref, tile0_sem,
        )
        tile0_copy.start()

        barrier_sem = pltpu.get_barrier_semaphore()
        for nbr in (y_partner, x_partner):
            pl.semaphore_signal(
                barrier_sem, inc=1,
                device_id=nbr, device_id_type=pl.DeviceIdType.MESH,
            )
        pl.semaphore_wait(barrier_sem, 2)

        tile0_copy.wait()
        y_rdmas = []
        for t in range(n_y):
            row = tile_row(t)
            src = tile0_ref if t == 0 else p_ref.at[pl.ds(peer_half + row, R), :]
            rdma = pltpu.make_async_remote_copy(
                src_ref=src,
                dst_ref=comm_ref.at[pl.ds(t * R, R), :],
                send_sem=y_send_sems.at[t],
                recv_sem=y_recv_sems.at[t],
                device_id=y_partner,
                device_id_type=pl.DeviceIdType.MESH,
            )
            rdma.start()
            y_rdmas.append(rdma)

        for c in local_copies:
            c.wait()
        g_row = g_ref[:].reshape(1, d)

        x_rdmas = []
        for t in range(n_y):
            row = tile_row(t)
            y_rdmas[t].wait_recv()
            acc = local_ref[pl.ds(t * R, R), :] + comm_ref[pl.ds(t * R, R), :]
            ms = jnp.mean(acc * acc, axis=-1, keepdims=True) + 1e-6
            out_ref[pl.ds(row, R), :] = acc * lax.rsqrt(ms) * g_row
            if t < n_push:
                rdma = pltpu.make_async_remote_copy(
                    src_ref=out_ref.at[pl.ds(row, R), :],
                    dst_ref=out_ref.at[pl.ds(row, R), :],
                    send_sem=x_send_sems.at[t],
                    recv_sem=x_recv_sems.at[t],
                    device_id=x_partner,
                    device_id_type=pl.DeviceIdType.MESH,
                )
                rdma.start()
                x_rdmas.append(rdma)

        for rdma in y_rdmas:
            rdma.wait_send()
        for rdma in x_rdmas:
            rdma.wait_send()
            rdma.wait_recv()

    return pl.pallas_call(
        body,
        out_shape=jax.ShapeDtypeStruct((half, d), jnp.float32),
        in_specs=[
            pl.BlockSpec(memory_space=pltpu.MemorySpace.HBM),
            pl.BlockSpec(memory_space=pltpu.VMEM),
        ],
        out_specs=pl.BlockSpec(memory_space=pltpu.VMEM),
        scratch_shapes=[
            pltpu.VMEM((chunk + K, d), jnp.float32),
            pltpu.VMEM((chunk + K, d), jnp.float32),
            pltpu.VMEM((R, d), jnp.float32),
            pltpu.SemaphoreType.DMA,
            pltpu.SemaphoreType.DMA,
            pltpu.SemaphoreType.DMA((n_y,)),
            pltpu.SemaphoreType.DMA((n_y,)),
            pltpu.SemaphoreType.DMA((n_push,)),
            pltpu.SemaphoreType.DMA((n_push,)),
        ],
        compiler_params=pltpu.CompilerParams(collective_id=0),
    )(partial, gamma)


# device time: 15371 ns/iter; 1.0240x vs baseline; 1.0240x over previous
import jax
import jax.numpy as jnp
from jax import lax
from jax.experimental import pallas as pl
from jax.experimental.pallas import tpu as pltpu

R = 32
K = 32


def kernel(partial, gamma):
    _, m, d = partial.shape
    half = m // 2
    chunk = half // 2
    n_mine = chunk // R
    n_late = K // R
    n_push = (chunk - K) // R
    n_y = n_mine + n_late

    def body(p3_ref, g_ref, out_ref, local_ref, comm_ref, tile0_ref,
             local_sem, tile0_sem,
             y_send_sems, y_recv_sems, x_send_sems, x_recv_sems):
        p_ref = p3_ref.at[0]
        my_x = lax.axis_index("x")
        my_y = lax.axis_index("y")
        my_z = lax.axis_index("z")
        y_partner = (my_x, 1 - my_y, my_z)
        x_partner = (1 - my_x, my_y, my_z)

        mine_half = my_y * half
        peer_half = (1 - my_y) * half
        my_chunk = my_x * chunk
        other_chunk = (1 - my_x) * chunk
        late = other_chunk + chunk - K

        def tile_row(t):
            if t < n_mine:
                return my_chunk + t * R
            return late + (t - n_mine) * R

        local_copies = [
            pltpu.make_async_copy(
                p_ref.at[pl.ds(mine_half + my_chunk, chunk), :],
                local_ref.at[pl.ds(0, chunk), :],
                local_sem,
            ),
            pltpu.make_async_copy(
                p_ref.at[pl.ds(mine_half + late, K), :],
                local_ref.at[pl.ds(chunk, K), :],
                local_sem,
            ),
        ]
        for c in local_copies:
            c.start()

        tile0_copy = pltpu.make_async_copy(
            p_ref.at[pl.ds(peer_half + tile_row(0), R), :],
            tile0_ref, tile0_sem,
        )
        tile0_copy.start()

        barrier_sem = pltpu.get_barrier_semaphore()
        for nbr in (y_partner, x_partner):
            pl.semaphore_signal(
                barrier_sem, inc=1,
                device_id=nbr, device_id_type=pl.DeviceIdType.MESH,
            )
        pl.semaphore_wait(barrier_sem, 2)

        tile0_copy.wait()
        y_rdmas = []
        for t in range(n_y):
            row = tile_row(t)
            src = tile0_ref if t == 0 else p_ref.at[pl.ds(peer_half + row, R), :]
            rdma = pltpu.make_async_remote_copy(
                src_ref=src,
                dst_ref=comm_ref.at[pl.ds(t * R, R), :],
                send_sem=y_send_sems.at[t],
                recv_sem=y_recv_sems.at[t],
                device_id=y_partner,
                device_id_type=pl.DeviceIdType.MESH,
            )
            rdma.start()
            y_rdmas.append(rdma)

        for c in local_copies:
            c.wait()
        g_row = g_ref[:].reshape(1, d)

        x_rdmas = []
        for t in range(n_y):
            row = tile_row(t)
            y_rdmas[t].wait_recv()
            acc = local_ref[pl.ds(t * R, R), :] + comm_ref[pl.ds(t * R, R), :]
            ms = jnp.mean(acc * acc, axis=-1, keepdims=True) + 1e-6
            out_ref[pl.ds(row, R), :] = acc * lax.rsqrt(ms) * g_row
            if t < n_push:
                rdma = pltpu.make_async_remote_copy(
                    src_ref=out_ref.at[pl.ds(row, R), :],
                    dst_ref=out_ref.at[pl.ds(row, R), :],
                    send_sem=x_send_sems.at[t],
                    recv_sem=x_recv_sems.at[t],
                    device_id=x_partner,
                    device_id_type=pl.DeviceIdType.MESH,
                )
                rdma.start()
                x_rdmas.append(rdma)

        for rdma in y_rdmas:
            rdma.wait_send()
        for rdma in x_rdmas:
            rdma.wait_send()
            rdma.wait_recv()

    return pl.pallas_call(
        body,
        out_shape=jax.ShapeDtypeStruct((half, d), jnp.float32),
        in_specs=[
            pl.BlockSpec(memory_space=pltpu.MemorySpace.HBM),
            pl.BlockSpec(memory_space=pltpu.VMEM),
        ],
        out_specs=pl.BlockSpec(memory_space=pltpu.VMEM),
        scratch_shapes=[
            pltpu.VMEM((chunk + K, d), jnp.float32),
            pltpu.VMEM((chunk + K, d), jnp.float32),
            pltpu.VMEM((R, d), jnp.float32),
            pltpu.SemaphoreType.DMA,
            pltpu.SemaphoreType.DMA,
            pltpu.SemaphoreType.DMA((n_y,)),
            pltpu.SemaphoreType.DMA((n_y,)),
            pltpu.SemaphoreType.DMA((n_push,)),
            pltpu.SemaphoreType.DMA((n_push,)),
        ],
        compiler_params=pltpu.CompilerParams(collective_id=0),
    )(partial, gamma)
